# baseline (speedup 1.0000x reference)
import functools

import jax
import jax.numpy as jnp
from jax import lax
from jax.experimental import pallas as pl
from jax.experimental.pallas import tpu as pltpu
from jax.experimental.pallas import tpu_sc as plsc

DX = (0.5, 0.5)
BX = (-49.75, -49.75)
BEV_DIM = (200, 200)

B = 256
T = 1800
MAP = 40000
L = 16
NC = 2
NS = 16
NW = NC * NS
BPW = B // NW
NCHUNK = (T + L - 1) // L
TPAD = NCHUNK * L


def _sc_body(trajs_hbm, c_hbm, out_hbm, c_v0, c_v1, tr_v, out_v, sem0, sem1):
    wid = lax.axis_index("s") * NC + lax.axis_index("c")
    base = wid * BPW
    c_bufs = (c_v0, c_v1)
    sems = (sem0, sem1)

    inv_dx = float(1.0 / DX[0])
    neg_bx = float(-BX[0])

    copy0 = pltpu.make_async_copy(c_hbm.at[base], c_bufs[0], sems[0])
    copy0.start()

    for k in range(BPW):
        b = base + k
        c_v = c_bufs[k % 2]
        pltpu.make_async_copy(c_hbm.at[b], c_v, sems[k % 2]).wait()
        if k + 1 < BPW:
            pltpu.make_async_copy(
                c_hbm.at[b + 1], c_bufs[(k + 1) % 2], sems[(k + 1) % 2]
            ).start()
        pltpu.sync_copy(trajs_hbm.at[b], tr_v)

        def chunk(i, _, c_v=c_v):
            t = jnp.minimum(i * L + lax.iota(jnp.int32, L), T - 1)
            xx = plsc.load_gather(tr_v, [t * 2])
            yy = plsc.load_gather(tr_v, [t * 2 + 1])
            xi = jnp.clip(((xx + neg_bx) * inv_dx).astype(jnp.int32),
                          0, BEV_DIM[0] - 1)
            yi = jnp.clip(((yy + neg_bx) * inv_dx).astype(jnp.int32),
                          0, BEV_DIM[1] - 1)
            out_v[pl.ds(i * L, L)] = plsc.load_gather(
                c_v, [xi * BEV_DIM[1] + yi])
            return 0

        lax.fori_loop(0, NCHUNK, chunk, 0)
        pltpu.sync_copy(out_v.at[pl.ds(0, T)], out_hbm.at[b])


@jax.jit
def kernel(trajs, C):
    trajs_flat = trajs.reshape(B, T * 2)
    c_flat = C.reshape(B, MAP)
    run = pl.kernel(
        _sc_body,
        out_type=jax.ShapeDtypeStruct((B, T), jnp.float32),
        mesh=plsc.VectorSubcoreMesh(
            core_axis_name="c", subcore_axis_name="s",
            num_cores=NC, num_subcores=NS),
        scratch_types=[
            pltpu.VMEM((MAP,), jnp.float32),
            pltpu.VMEM((MAP,), jnp.float32),
            pltpu.VMEM((T * 2,), jnp.float32),
            pltpu.VMEM((TPAD,), jnp.float32),
            pltpu.SemaphoreType.DMA,
            pltpu.SemaphoreType.DMA,
        ],
        compiler_params=pltpu.CompilerParams(
            needs_layout_passes=False, use_tc_tiling_on_sc=False),
    )
    return run(trajs_flat, c_flat)
